# speedup vs baseline: 1.0213x; 1.0213x over previous
import functools

import jax
import jax.numpy as jnp
from jax import lax
from jax.experimental import pallas as pl
from jax.experimental.pallas import tpu as pltpu
from jax.experimental.pallas import tpu_sc as plsc

BATCH = 4
SEQ_LEN = 8192
EMBD_DIM = 128
NUM_TOKENS = BATCH * SEQ_LEN

NUM_CORES = 2
NUM_SUBCORES = 16
NW = NUM_CORES * NUM_SUBCORES
TOK_PER_W = NUM_TOKENS // NW
CHUNK = 128
NCHUNK = TOK_PER_W // CHUNK
LANES = 16


def _sc_body(x_hbm, embd_hbm, pos_hbm, out_hbm, idx_v, rows_v, pos_v, gsem):
    wid = lax.axis_index("s") * NUM_CORES + lax.axis_index("c")
    base = wid * TOK_PER_W
    t0 = (wid % (SEQ_LEN // TOK_PER_W)) * TOK_PER_W

    pltpu.sync_copy(x_hbm.at[wid], idx_v)

    for c in range(NCHUNK):
        cp = pltpu.async_copy(embd_hbm.at[idx_v.at[c]], rows_v, gsem)
        pltpu.sync_copy(pos_hbm.at[pl.ds(t0 + c * CHUNK, CHUNK)], pos_v)
        cp.wait()

        def add_row(r, _):
            for j in range(EMBD_DIM // LANES):
                s = pl.ds(j * LANES, LANES)
                rows_v[r, s] = rows_v[r, s] + pos_v[r, s]
            return 0

        lax.fori_loop(0, CHUNK, add_row, 0)

        pltpu.sync_copy(rows_v, out_hbm.at[pl.ds(base + c * CHUNK, CHUNK)])


@jax.jit
def kernel(x, embd_table, pos_table):
    xr = x.astype(jnp.int32).reshape(NW, NCHUNK, CHUNK)
    mesh = plsc.VectorSubcoreMesh(core_axis_name="c", subcore_axis_name="s")
    out = pl.kernel(
        _sc_body,
        out_type=jax.ShapeDtypeStruct((NUM_TOKENS, EMBD_DIM), jnp.float32),
        mesh=mesh,
        scratch_types=[
            pltpu.VMEM((NCHUNK, CHUNK), jnp.int32),
            pltpu.VMEM((CHUNK, EMBD_DIM), jnp.float32),
            pltpu.VMEM((CHUNK, EMBD_DIM), jnp.float32),
            pltpu.SemaphoreType.DMA,
        ],
    )(xr, embd_table, pos_table)
    return out.reshape(BATCH, SEQ_LEN, EMBD_DIM)


# speedup vs baseline: 1.0799x; 1.0573x over previous
import functools

import jax
import jax.numpy as jnp
from jax import lax
from jax.experimental import pallas as pl
from jax.experimental.pallas import tpu as pltpu
from jax.experimental.pallas import tpu_sc as plsc

BATCH = 4
SEQ_LEN = 8192
EMBD_DIM = 128
NUM_TOKENS = BATCH * SEQ_LEN

NUM_CORES = 2
NUM_SUBCORES = 16
NW = NUM_CORES * NUM_SUBCORES
TOK_PER_W = NUM_TOKENS // NW
CHUNK = 128
NCHUNK = TOK_PER_W // CHUNK
LANES = 16


def _sc_body(x_hbm, embd_hbm, pos_hbm, out_hbm, idx_v, rows_v, pos_v, gsem):
    wid = lax.axis_index("s") * NUM_CORES + lax.axis_index("c")
    base = wid * TOK_PER_W
    t0 = (wid % (SEQ_LEN // TOK_PER_W)) * TOK_PER_W

    pltpu.sync_copy(x_hbm.at[wid], idx_v)

    for c in range(NCHUNK):
        pltpu.sync_copy(pos_hbm.at[pl.ds(t0 + c * CHUNK, CHUNK)], rows_v)
        pltpu.async_copy(embd_hbm.at[idx_v.at[c]], rows_v, gsem, add=True).wait()
        pltpu.sync_copy(rows_v, out_hbm.at[pl.ds(base + c * CHUNK, CHUNK)])


@jax.jit
def kernel(x, embd_table, pos_table):
    xr = x.astype(jnp.int32).reshape(NW, NCHUNK, CHUNK)
    mesh = plsc.VectorSubcoreMesh(core_axis_name="c", subcore_axis_name="s")
    out = pl.kernel(
        _sc_body,
        out_type=jax.ShapeDtypeStruct((NUM_TOKENS, EMBD_DIM), jnp.float32),
        mesh=mesh,
        scratch_types=[
            pltpu.VMEM((NCHUNK, CHUNK), jnp.int32),
            pltpu.VMEM((CHUNK, EMBD_DIM), jnp.float32),
            pltpu.VMEM((CHUNK, EMBD_DIM), jnp.float32),
            pltpu.SemaphoreType.DMA,
        ],
    )(xr, embd_table, pos_table)
    return out.reshape(BATCH, SEQ_LEN, EMBD_DIM)


# speedup vs baseline: 1.3578x; 1.2573x over previous
import functools

import jax
import jax.numpy as jnp
from jax import lax
from jax.experimental import pallas as pl
from jax.experimental.pallas import tpu as pltpu
from jax.experimental.pallas import tpu_sc as plsc

BATCH = 4
SEQ_LEN = 8192
EMBD_DIM = 128
NUM_TOKENS = BATCH * SEQ_LEN

NUM_CORES = 2
NUM_SUBCORES = 16
NW = NUM_CORES * NUM_SUBCORES
TOK_PER_W = NUM_TOKENS // NW
CHUNK = 128
NCHUNK = TOK_PER_W // CHUNK
LANES = 16
NBUF = 6
LEAD = 3


def _sc_body(x_hbm, embd_hbm, pos_hbm, out_hbm, idx_v, *scratch):
    bufs = scratch[:NBUF]
    psems = scratch[NBUF:2 * NBUF]
    osems = scratch[2 * NBUF:3 * NBUF]
    gsems = scratch[3 * NBUF:]

    wid = lax.axis_index("s") * NUM_CORES + lax.axis_index("c")
    base = wid * TOK_PER_W
    t0 = (wid % (SEQ_LEN // TOK_PER_W)) * TOK_PER_W

    pltpu.sync_copy(x_hbm.at[wid], idx_v)

    def pos_start(c):
        b = c % NBUF
        return pltpu.async_copy(
            pos_hbm.at[pl.ds(t0 + c * CHUNK, CHUNK)], bufs[b], psems[b])

    def gather_start(c):
        b = c % NBUF
        return pltpu.async_copy(
            embd_hbm.at[idx_v.at[c]], bufs[b], gsems[c % 2], add=True)

    def store_start(c):
        b = c % NBUF
        return pltpu.async_copy(
            bufs[b], out_hbm.at[pl.ds(base + c * CHUNK, CHUNK)], osems[b])

    pos_cp = [None] * NCHUNK
    g_cp = [None] * NCHUNK
    s_cp = [None] * NCHUNK
    s_waited = [False] * NCHUNK

    for c in range(min(LEAD + 1, NCHUNK)):
        pos_cp[c] = pos_start(c)
    for c in range(NCHUNK):
        nxt = c + LEAD
        if nxt < NCHUNK and pos_cp[nxt] is None:
            prev = nxt - NBUF
            if prev >= 0 and not s_waited[prev]:
                s_cp[prev].wait()
                s_waited[prev] = True
            pos_cp[nxt] = pos_start(nxt)
        pos_cp[c].wait()
        g_cp[c] = gather_start(c)
        if c >= 1:
            g_cp[c - 1].wait()
            s_cp[c - 1] = store_start(c - 1)
    g_cp[NCHUNK - 1].wait()
    s_cp[NCHUNK - 1] = store_start(NCHUNK - 1)
    for c in range(NCHUNK):
        if not s_waited[c]:
            s_cp[c].wait()
            s_waited[c] = True


@jax.jit
def kernel(x, embd_table, pos_table):
    xr = x.astype(jnp.int32).reshape(NW, NCHUNK, CHUNK)
    mesh = plsc.VectorSubcoreMesh(core_axis_name="c", subcore_axis_name="s")
    out = pl.kernel(
        _sc_body,
        out_type=jax.ShapeDtypeStruct((NUM_TOKENS, EMBD_DIM), jnp.float32),
        mesh=mesh,
        scratch_types=(
            [pltpu.VMEM((NCHUNK, CHUNK), jnp.int32)]
            + [pltpu.VMEM((CHUNK, EMBD_DIM), jnp.float32)] * NBUF
            + [pltpu.SemaphoreType.DMA] * (2 * NBUF + 2)
        ),
    )(xr, embd_table, pos_table)
    return out.reshape(BATCH, SEQ_LEN, EMBD_DIM)


# speedup vs baseline: 1.3591x; 1.0010x over previous
import functools

import jax
import jax.numpy as jnp
from jax import lax
from jax.experimental import pallas as pl
from jax.experimental.pallas import tpu as pltpu
from jax.experimental.pallas import tpu_sc as plsc

BATCH = 4
SEQ_LEN = 8192
EMBD_DIM = 128
NUM_TOKENS = BATCH * SEQ_LEN

NUM_CORES = 2
NUM_SUBCORES = 16
NW = NUM_CORES * NUM_SUBCORES
TOK_PER_W = NUM_TOKENS // NW
CHUNK = 128
NCHUNK = TOK_PER_W // CHUNK
LANES = 16
NBUF = 6
LEAD = 3


def _sc_body(x_hbm, embd_hbm, pos_hbm, out_hbm, idx_v, *scratch):
    bufs = scratch[:NBUF]
    psems = scratch[NBUF:2 * NBUF]
    osems = scratch[2 * NBUF:3 * NBUF]
    gsems = scratch[3 * NBUF:]

    wid = lax.axis_index("s") * NUM_CORES + lax.axis_index("c")
    base = wid * TOK_PER_W
    t0 = (wid % (SEQ_LEN // TOK_PER_W)) * TOK_PER_W

    pltpu.sync_copy(x_hbm.at[wid], idx_v)

    def pos_start(c):
        b = c % NBUF
        return pltpu.async_copy(
            pos_hbm.at[pl.ds(t0 + c * CHUNK, CHUNK)], bufs[b], psems[b])

    def gather_start(c):
        b = c % NBUF
        return pltpu.async_copy(
            embd_hbm.at[idx_v.at[c]], bufs[b], gsems[c % 3], add=True)

    def store_start(c):
        b = c % NBUF
        return pltpu.async_copy(
            bufs[b], out_hbm.at[pl.ds(base + c * CHUNK, CHUNK)], osems[b])

    pos_cp = [None] * NCHUNK
    g_cp = [None] * NCHUNK
    s_cp = [None] * NCHUNK
    s_waited = [False] * NCHUNK

    for c in range(min(LEAD + 1, NCHUNK)):
        pos_cp[c] = pos_start(c)
    for c in range(NCHUNK):
        nxt = c + LEAD
        if nxt < NCHUNK and pos_cp[nxt] is None:
            prev = nxt - NBUF
            if prev >= 0 and not s_waited[prev]:
                s_cp[prev].wait()
                s_waited[prev] = True
            pos_cp[nxt] = pos_start(nxt)
        pos_cp[c].wait()
        g_cp[c] = gather_start(c)
        if c >= 2:
            g_cp[c - 2].wait()
            s_cp[c - 2] = store_start(c - 2)
    for c in range(max(0, NCHUNK - 2), NCHUNK):
        g_cp[c].wait()
        s_cp[c] = store_start(c)
    for c in range(NCHUNK):
        if not s_waited[c]:
            s_cp[c].wait()
            s_waited[c] = True


@jax.jit
def kernel(x, embd_table, pos_table):
    xr = x.astype(jnp.int32).reshape(NW, NCHUNK, CHUNK)
    mesh = plsc.VectorSubcoreMesh(core_axis_name="c", subcore_axis_name="s")
    out = pl.kernel(
        _sc_body,
        out_type=jax.ShapeDtypeStruct((NUM_TOKENS, EMBD_DIM), jnp.float32),
        mesh=mesh,
        scratch_types=(
            [pltpu.VMEM((NCHUNK, CHUNK), jnp.int32)]
            + [pltpu.VMEM((CHUNK, EMBD_DIM), jnp.float32)] * NBUF
            + [pltpu.SemaphoreType.DMA] * (2 * NBUF + 3)
        ),
    )(xr, embd_table, pos_table)
    return out.reshape(BATCH, SEQ_LEN, EMBD_DIM)


# speedup vs baseline: 1.4020x; 1.0316x over previous
import functools

import jax
import jax.numpy as jnp
from jax import lax
from jax.experimental import pallas as pl
from jax.experimental.pallas import tpu as pltpu
from jax.experimental.pallas import tpu_sc as plsc

BATCH = 4
SEQ_LEN = 8192
EMBD_DIM = 128
NUM_TOKENS = BATCH * SEQ_LEN

NUM_CORES = 2
NUM_SUBCORES = 16
NW = NUM_CORES * NUM_SUBCORES
TOK_PER_W = NUM_TOKENS // NW
CHUNK = 128
NCHUNK = TOK_PER_W // CHUNK
LANES = 16
NBUF = 5
DEPTH = 3
TROWS = SEQ_LEN // NW


def _sc_body(x_hbm, embd_hbm, pos_hbm, out_hbm, idx_v, pos_v, *scratch):
    bufs = scratch[:NBUF]
    osems = scratch[NBUF:2 * NBUF]
    gsems = scratch[2 * NBUF:]

    wid = lax.axis_index("s") * NUM_CORES + lax.axis_index("c")
    t0 = wid * TROWS

    pltpu.sync_copy(pos_hbm.at[pl.ds(t0, TROWS)], pos_v)
    pltpu.sync_copy(x_hbm.at[wid], idx_v)

    def gather_start(c):
        b = c % NBUF
        return pltpu.async_copy(
            embd_hbm.at[idx_v.at[c]], bufs[b], gsems[c % (DEPTH + 1)])

    def store_start(c):
        b = c % NBUF
        base = (c // 2) * SEQ_LEN + t0 + (c % 2) * CHUNK
        return pltpu.async_copy(
            bufs[b], out_hbm.at[pl.ds(base, CHUNK)], osems[b])

    def add_pos(c):
        buf = bufs[c % NBUF]
        po = (c % 2) * CHUNK

        @plsc.parallel_loop(0, CHUNK, unroll=4)
        def _(r):
            for j in range(EMBD_DIM // LANES):
                s = pl.ds(j * LANES, LANES)
                buf[r, s] = buf[r, s] + pos_v[po + r, s]

    g_cp = [None] * NCHUNK
    s_cp = [None] * NCHUNK
    s_waited = [False] * NCHUNK

    for c in range(min(DEPTH, NCHUNK)):
        g_cp[c] = gather_start(c)
    for c in range(NCHUNK):
        nxt = c + DEPTH
        if nxt < NCHUNK:
            prev = nxt - NBUF
            if prev >= 0 and not s_waited[prev]:
                s_cp[prev].wait()
                s_waited[prev] = True
            g_cp[nxt] = gather_start(nxt)
        g_cp[c].wait()
        add_pos(c)
        s_cp[c] = store_start(c)
    for c in range(NCHUNK):
        if not s_waited[c]:
            s_cp[c].wait()
            s_waited[c] = True


@jax.jit
def kernel(x, embd_table, pos_table):
    xr = (x.astype(jnp.int32)
          .reshape(BATCH, NW, 2, CHUNK)
          .transpose(1, 0, 2, 3)
          .reshape(NW, NCHUNK, CHUNK))
    mesh = plsc.VectorSubcoreMesh(core_axis_name="c", subcore_axis_name="s")
    out = pl.kernel(
        _sc_body,
        out_type=jax.ShapeDtypeStruct((NUM_TOKENS, EMBD_DIM), jnp.float32),
        mesh=mesh,
        scratch_types=(
            [pltpu.VMEM((NCHUNK, CHUNK), jnp.int32),
             pltpu.VMEM((TROWS, EMBD_DIM), jnp.float32)]
            + [pltpu.VMEM((CHUNK, EMBD_DIM), jnp.float32)] * NBUF
            + [pltpu.SemaphoreType.DMA] * (NBUF + DEPTH + 1)
        ),
    )(xr, embd_table, pos_table)
    return out.reshape(BATCH, SEQ_LEN, EMBD_DIM)
